# baseline (speedup 1.0000x reference)
import functools

import jax
import jax.numpy as jnp
import numpy as np
from jax import lax
from jax.experimental import pallas as pl
from jax.experimental.pallas import tpu as pltpu
from jax.experimental.pallas import tpu_sc as plsc

N = 10000
D = 128
E = 320000
NC = 2
NS = 16
NW = NC * NS
C = 64
NCH = 160
G = 32
NB = 4
EPT = NCH * C
EP = NW * EPT
RPT = N // NS

_mesh = plsc.VectorSubcoreMesh(core_axis_name="c", subcore_axis_name="s")
_sc_params = pltpu.CompilerParams(
    needs_layout_passes=False, use_tc_tiling_on_sc=False)


@functools.partial(
    pl.kernel,
    out_type=jax.ShapeDtypeStruct((NW, N), jnp.float32),
    mesh=_mesh,
    compiler_params=_sc_params,
    scratch_types=[
        pltpu.VMEM((EPT // 16, 16), jnp.int32),
        pltpu.VMEM((EPT // 16, 16), jnp.float32),
        pltpu.VMEM((N,), jnp.float32),
    ],
)
def _deg_kernel(col_h, w_h, out_h, col_v, w_v, deg_v):
    c = lax.axis_index("c")
    s = lax.axis_index("s")
    wid = c * NS + s
    zv = jnp.zeros((16,), jnp.float32)

    def zero(i, _):
        deg_v[pl.ds(i * 16, 16)] = zv
        return 0

    lax.fori_loop(0, N // 16, zero, 0)
    pltpu.sync_copy(col_h.at[wid], col_v)
    pltpu.sync_copy(w_h.at[wid], w_v)

    def body(i, _):
        plsc.addupdate_scatter(deg_v, [col_v[i]], w_v[i])
        return 0

    lax.fori_loop(0, EPT // 16, body, 0)
    pltpu.sync_copy(deg_v, out_h.at[wid])


@functools.partial(
    pl.kernel,
    out_type=jax.ShapeDtypeStruct((NC, N, D), jnp.float32),
    mesh=_mesh,
    compiler_params=_sc_params,
    scratch_types=[
        pltpu.VMEM((G, C), jnp.int32),
        pltpu.VMEM((G, C), jnp.int32),
        pltpu.VMEM((G, C), jnp.float32),
        [pltpu.VMEM((C, D), jnp.bfloat16)] * NB,
        [pltpu.VMEM((C, D), jnp.float32)] * 2,
        pltpu.VMEM_SHARED((N, D), jnp.float32),
        [pltpu.SemaphoreType.DMA] * NB,
        [pltpu.SemaphoreType.DMA] * 2,
    ],
)
def _agg_kernel(row_h, col_h, w_h, y_h, out_h,
                row_v, col_v, w_v, hbufs, sbufs, acc, gsems, ssems):
    c = lax.axis_index("c")
    s = lax.axis_index("s")
    wid = c * NS + s
    zv = jnp.zeros((16,), jnp.float32)

    def zrow(i, _):
        for j in range(D // 16):
            sbufs[0][i, pl.ds(j * 16, 16)] = zv
        return 0

    lax.fori_loop(0, C, zrow, 0)

    def zacc(k, _):
        pltpu.sync_copy(sbufs[0], acc.at[pl.ds(s * RPT + k * C, C)])
        return 0

    lax.fori_loop(0, RPT // C, zacc, 0)
    pltpu.sync_copy(sbufs[0].at[pl.ds(0, RPT % C)],
                    acc.at[pl.ds(s * RPT + (RPT // C) * C, RPT % C)])
    plsc.subcore_barrier()

    himm = jnp.full((16,), -65536, jnp.int32)

    def cscale(hbuf, sbuf, j):
        def sgrp(g16, _):
            wv = w_v[j, pl.ds(g16 * 16, 16)]
            for l in range(16):
                wb = jnp.take_along_axis(
                    wv, jnp.full((16,), l, jnp.int32), axis=0)
                r = g16 * 16 + l
                for k in range(D // 32):
                    u = plsc.bitcast(hbuf[r, pl.ds(k * 32, 32)], jnp.int32)
                    flo = plsc.bitcast(u << 16, jnp.float32)
                    fhi = plsc.bitcast(u & himm, jnp.float32)
                    sbuf[r, pl.ds(k * 32, 16)] = flo * wb
                    sbuf[r, pl.ds(k * 32 + 16, 16)] = fhi * wb
            return 0

        lax.fori_loop(0, C // 16, sgrp, 0)

    def group(grp, _):
        pltpu.sync_copy(row_h.at[wid, pl.ds(grp * G, G)], row_v)
        pltpu.sync_copy(col_h.at[wid, pl.ds(grp * G, G)], col_v)
        pltpu.sync_copy(w_h.at[wid, pl.ds(grp * G, G)], w_v)
        for b in range(NB):
            pltpu.async_copy(y_h.at[row_v.at[b]], hbufs[b], gsems[b])

        def chunk4(q, _):
            for b in range(NB):
                j = q * NB + b
                t = b % 2
                pltpu.make_async_copy(
                    y_h.at[row_v.at[j]], hbufs[b], gsems[b]).wait()

                @pl.when(j >= 2)
                def _():
                    pltpu.make_async_copy(
                        sbufs[t], acc.at[col_v.at[0]], ssems[t]).wait()

                cscale(hbufs[b], sbufs[t], j)
                pltpu.async_copy(
                    sbufs[t], acc.at[col_v.at[j]], ssems[t], add=True)
                j4 = j + NB

                @pl.when(j4 < G)
                def _():
                    pltpu.async_copy(
                        y_h.at[row_v.at[j4]], hbufs[b], gsems[b])

            return 0

        lax.fori_loop(0, G // NB, chunk4, 0)
        pltpu.make_async_copy(
            sbufs[0], acc.at[col_v.at[0]], ssems[0]).wait()
        pltpu.make_async_copy(
            sbufs[1], acc.at[col_v.at[0]], ssems[1]).wait()
        return 0

    lax.fori_loop(0, NCH // G, group, 0)
    plsc.subcore_barrier()
    pltpu.sync_copy(acc.at[pl.ds(s * RPT, RPT)],
                    out_h.at[c, pl.ds(s * RPT, RPT)])


_BLK = 1000
_GRID = N // _BLK


def _shuffle(y):
    p = lax.broadcasted_iota(jnp.int32, y.shape, 1)
    q = p % 32
    idx = 32 * (p // 32) + q // 2 + 16 * (q % 2)
    return jnp.take_along_axis(y, idx, axis=1).astype(jnp.bfloat16)


def _tc0_body(degp_ref, dinv_ref):
    deg = jnp.sum(degp_ref[...], axis=0) + 1.0
    dinv_ref[...] = lax.rsqrt(deg)[:, None]


def _tc1_body(x_ref, w_ref, dinv_ref, y_ref, ysh_ref):
    xw = jnp.dot(x_ref[...], w_ref[...], preferred_element_type=jnp.float32)
    y = xw * dinv_ref[...]
    y_ref[...] = y
    ysh_ref[...] = _shuffle(y)


def _tc2_body(p_ref, y1_ref, dinv_ref, b1_ref, w2_ref, y2_ref, y2sh_ref):
    dinv = dinv_ref[...]
    agg = p_ref[0] + p_ref[1] + y1_ref[...]
    h = jnp.maximum(agg * dinv + b1_ref[...][None, :], 0.0)
    hw = jnp.dot(h, w2_ref[...], preferred_element_type=jnp.float32)
    y2 = hw * dinv
    y2_ref[...] = y2
    y2sh_ref[...] = _shuffle(y2)


def _tc3_body(q_ref, y2_ref, dinv_ref, b2_ref, out_ref):
    agg = q_ref[0] + q_ref[1] + y2_ref[...]
    out_ref[...] = jnp.maximum(
        agg * dinv_ref[...] + b2_ref[...][None, :], 0.0)


_tc0 = pl.pallas_call(
    _tc0_body,
    out_shape=jax.ShapeDtypeStruct((N, 1), jnp.float32),
)

_tc1 = pl.pallas_call(
    _tc1_body,
    grid=(_GRID,),
    in_specs=[
        pl.BlockSpec((_BLK, D), lambda i: (i, 0)),
        pl.BlockSpec((D, D), lambda i: (0, 0)),
        pl.BlockSpec((_BLK, 1), lambda i: (i, 0)),
    ],
    out_specs=[
        pl.BlockSpec((_BLK, D), lambda i: (i, 0)),
        pl.BlockSpec((_BLK, D), lambda i: (i, 0)),
    ],
    out_shape=[
        jax.ShapeDtypeStruct((N, D), jnp.float32),
        jax.ShapeDtypeStruct((N, D), jnp.bfloat16),
    ],
)

_tc2 = pl.pallas_call(
    _tc2_body,
    grid=(_GRID,),
    in_specs=[
        pl.BlockSpec((NC, _BLK, D), lambda i: (0, i, 0)),
        pl.BlockSpec((_BLK, D), lambda i: (i, 0)),
        pl.BlockSpec((_BLK, 1), lambda i: (i, 0)),
        pl.BlockSpec((D,), lambda i: (0,)),
        pl.BlockSpec((D, D), lambda i: (0, 0)),
    ],
    out_specs=[
        pl.BlockSpec((_BLK, D), lambda i: (i, 0)),
        pl.BlockSpec((_BLK, D), lambda i: (i, 0)),
    ],
    out_shape=[
        jax.ShapeDtypeStruct((N, D), jnp.float32),
        jax.ShapeDtypeStruct((N, D), jnp.bfloat16),
    ],
)

_tc3 = pl.pallas_call(
    _tc3_body,
    grid=(_GRID,),
    in_specs=[
        pl.BlockSpec((NC, _BLK, D), lambda i: (0, i, 0)),
        pl.BlockSpec((_BLK, D), lambda i: (i, 0)),
        pl.BlockSpec((_BLK, 1), lambda i: (i, 0)),
        pl.BlockSpec((D,), lambda i: (0,)),
    ],
    out_specs=pl.BlockSpec((_BLK, D), lambda i: (i, 0)),
    out_shape=jax.ShapeDtypeStruct((N, D), jnp.float32),
)


@jax.jit
def kernel(x, edge_index, edge_weight, W1, b1, W2, b2):
    pad = EP - E
    pad_idx = jnp.asarray((np.arange(pad) * 13) % N, dtype=jnp.int32)
    row = jnp.concatenate([edge_index[0], pad_idx]).reshape(NW, NCH, C)
    col = jnp.concatenate([edge_index[1], pad_idx]).reshape(NW, NCH, C)
    w = jnp.pad(edge_weight, (0, pad)).reshape(NW, NCH, C)
    col16 = col.reshape(NW, EPT // 16, 16)
    w16 = w.reshape(NW, EPT // 16, 16)

    degp = _deg_kernel(col16, w16)
    dinv = _tc0(degp)
    y1, y1sh = _tc1(x, W1, dinv)
    p = _agg_kernel(row, col, w, y1sh)
    y2, y2sh = _tc2(p, y1, dinv, b1, W2)
    q = _agg_kernel(row, col, w, y2sh)
    out = _tc3(q, y2, dinv, b2)
    return out


# speedup vs baseline: 1.8168x; 1.8168x over previous
import functools

import jax
import jax.numpy as jnp
import numpy as np
from jax import lax
from jax.experimental import pallas as pl
from jax.experimental.pallas import tpu as pltpu
from jax.experimental.pallas import tpu_sc as plsc

N = 10000
D = 128
E = 320000
NC = 2
NS = 16
NW = NC * NS
C = 64
NCH = 160
G = 32
NB = 4
EPT = NCH * C
EP = NW * EPT
RPT = N // NS

_mesh = plsc.VectorSubcoreMesh(core_axis_name="c", subcore_axis_name="s")
_sc_params = pltpu.CompilerParams(
    needs_layout_passes=False, use_tc_tiling_on_sc=False)


@functools.partial(
    pl.kernel,
    out_type=jax.ShapeDtypeStruct((NW, N), jnp.float32),
    mesh=_mesh,
    compiler_params=_sc_params,
    scratch_types=[
        pltpu.VMEM((EPT // 16, 16), jnp.int32),
        pltpu.VMEM((EPT // 16, 16), jnp.float32),
        pltpu.VMEM((N,), jnp.float32),
    ],
)
def _deg_kernel(col_h, w_h, out_h, col_v, w_v, deg_v):
    c = lax.axis_index("c")
    s = lax.axis_index("s")
    wid = c * NS + s
    zv = jnp.zeros((16,), jnp.float32)

    def zero(i, _):
        deg_v[pl.ds(i * 16, 16)] = zv
        return 0

    lax.fori_loop(0, N // 16, zero, 0)
    pltpu.sync_copy(col_h.at[wid], col_v)
    pltpu.sync_copy(w_h.at[wid], w_v)

    def body(i, _):
        plsc.addupdate_scatter(deg_v, [col_v[i]], w_v[i])
        return 0

    lax.fori_loop(0, EPT // 16, body, 0)
    pltpu.sync_copy(deg_v, out_h.at[wid])


@functools.partial(
    pl.kernel,
    out_type=jax.ShapeDtypeStruct((NC, N, D), jnp.float32),
    mesh=_mesh,
    compiler_params=_sc_params,
    scratch_types=[
        pltpu.VMEM((G, C), jnp.int32),
        pltpu.VMEM((G, C), jnp.int32),
        pltpu.VMEM((G, C), jnp.float32),
        [pltpu.VMEM((C, D), jnp.float32)] * NB,
        pltpu.VMEM_SHARED((N, D), jnp.float32),
        [pltpu.SemaphoreType.DMA] * NB,
        [pltpu.SemaphoreType.DMA] * NB,
    ],
)
def _agg_kernel(row_h, col_h, w_h, y_h, out_h,
                row_v, col_v, w_v, bufs, acc, gsems, ssems):
    c = lax.axis_index("c")
    s = lax.axis_index("s")
    wid = c * NS + s
    zv = jnp.zeros((16,), jnp.float32)

    def zrow(i, _):
        for j in range(D // 16):
            bufs[0][i, pl.ds(j * 16, 16)] = zv
        return 0

    lax.fori_loop(0, C, zrow, 0)

    def zacc(k, _):
        pltpu.sync_copy(bufs[0], acc.at[pl.ds(s * RPT + k * C, C)])
        return 0

    lax.fori_loop(0, RPT // C, zacc, 0)
    pltpu.sync_copy(bufs[0].at[pl.ds(0, RPT % C)],
                    acc.at[pl.ds(s * RPT + (RPT // C) * C, RPT % C)])
    plsc.subcore_barrier()

    def scale(buf, j):
        def sgrp(g16, _):
            wv = w_v[j, pl.ds(g16 * 16, 16)]
            for l in range(16):
                wb = jnp.take_along_axis(
                    wv, jnp.full((16,), l, jnp.int32), axis=0)
                for k in range(D // 16):
                    sl = pl.ds(k * 16, 16)
                    buf[g16 * 16 + l, sl] = buf[g16 * 16 + l, sl] * wb
            return 0

        lax.fori_loop(0, C // 16, sgrp, 0)

    def group(grp, _):
        pltpu.sync_copy(row_h.at[wid, pl.ds(grp * G, G)], row_v)
        pltpu.sync_copy(col_h.at[wid, pl.ds(grp * G, G)], col_v)
        pltpu.sync_copy(w_h.at[wid, pl.ds(grp * G, G)], w_v)
        for b in range(2):
            pltpu.async_copy(y_h.at[row_v.at[b]], bufs[b], gsems[b])

        def chunk4(q, _):
            for b in range(NB):
                j = q * NB + b
                buf = bufs[b]
                pltpu.make_async_copy(
                    y_h.at[row_v.at[j]], buf, gsems[b]).wait()
                scale(buf, j)
                pltpu.async_copy(
                    buf, acc.at[col_v.at[j]], ssems[b], add=True)
                b2 = (b + 2) % NB
                j2 = j + 2

                @pl.when(j >= 2)
                def _():
                    pltpu.make_async_copy(
                        bufs[b2], acc.at[col_v.at[0]], ssems[b2]).wait()

                @pl.when(j2 < G)
                def _():
                    pltpu.async_copy(
                        y_h.at[row_v.at[j2]], bufs[b2], gsems[b2])

            return 0

        lax.fori_loop(0, G // NB, chunk4, 0)
        pltpu.make_async_copy(
            bufs[2], acc.at[col_v.at[0]], ssems[2]).wait()
        pltpu.make_async_copy(
            bufs[3], acc.at[col_v.at[0]], ssems[3]).wait()
        return 0

    lax.fori_loop(0, NCH // G, group, 0)
    plsc.subcore_barrier()
    pltpu.sync_copy(acc.at[pl.ds(s * RPT, RPT)],
                    out_h.at[c, pl.ds(s * RPT, RPT)])


_BLK = 1000
_GRID = N // _BLK


def _tc0_body(degp_ref, dinv_ref):
    deg = jnp.sum(degp_ref[...], axis=0) + 1.0
    dinv_ref[...] = lax.rsqrt(deg)[:, None]


def _tc1_body(x_ref, w_ref, dinv_ref, y_ref):
    xw = jnp.dot(x_ref[...], w_ref[...], preferred_element_type=jnp.float32)
    y_ref[...] = xw * dinv_ref[...]


def _tc2_body(p_ref, y1_ref, dinv_ref, b1_ref, w2_ref, y2_ref):
    dinv = dinv_ref[...]
    agg = p_ref[0] + p_ref[1] + y1_ref[...]
    h = jnp.maximum(agg * dinv + b1_ref[...][None, :], 0.0)
    hw = jnp.dot(h, w2_ref[...], preferred_element_type=jnp.float32)
    y2_ref[...] = hw * dinv


def _tc3_body(q_ref, y2_ref, dinv_ref, b2_ref, out_ref):
    agg = q_ref[0] + q_ref[1] + y2_ref[...]
    out_ref[...] = jnp.maximum(
        agg * dinv_ref[...] + b2_ref[...][None, :], 0.0)


_tc0 = pl.pallas_call(
    _tc0_body,
    out_shape=jax.ShapeDtypeStruct((N, 1), jnp.float32),
)

_tc1 = pl.pallas_call(
    _tc1_body,
    grid=(_GRID,),
    in_specs=[
        pl.BlockSpec((_BLK, D), lambda i: (i, 0)),
        pl.BlockSpec((D, D), lambda i: (0, 0)),
        pl.BlockSpec((_BLK, 1), lambda i: (i, 0)),
    ],
    out_specs=pl.BlockSpec((_BLK, D), lambda i: (i, 0)),
    out_shape=jax.ShapeDtypeStruct((N, D), jnp.float32),
)

_tc2 = pl.pallas_call(
    _tc2_body,
    grid=(_GRID,),
    in_specs=[
        pl.BlockSpec((NC, _BLK, D), lambda i: (0, i, 0)),
        pl.BlockSpec((_BLK, D), lambda i: (i, 0)),
        pl.BlockSpec((_BLK, 1), lambda i: (i, 0)),
        pl.BlockSpec((D,), lambda i: (0,)),
        pl.BlockSpec((D, D), lambda i: (0, 0)),
    ],
    out_specs=pl.BlockSpec((_BLK, D), lambda i: (i, 0)),
    out_shape=jax.ShapeDtypeStruct((N, D), jnp.float32),
)

_tc3 = pl.pallas_call(
    _tc3_body,
    grid=(_GRID,),
    in_specs=[
        pl.BlockSpec((NC, _BLK, D), lambda i: (0, i, 0)),
        pl.BlockSpec((_BLK, D), lambda i: (i, 0)),
        pl.BlockSpec((_BLK, 1), lambda i: (i, 0)),
        pl.BlockSpec((D,), lambda i: (0,)),
    ],
    out_specs=pl.BlockSpec((_BLK, D), lambda i: (i, 0)),
    out_shape=jax.ShapeDtypeStruct((N, D), jnp.float32),
)


@jax.jit
def kernel(x, edge_index, edge_weight, W1, b1, W2, b2):
    pad = EP - E
    pad_idx = jnp.asarray((np.arange(pad) * 13) % N, dtype=jnp.int32)
    row = jnp.concatenate([edge_index[0], pad_idx]).reshape(NW, NCH, C)
    col = jnp.concatenate([edge_index[1], pad_idx]).reshape(NW, NCH, C)
    w = jnp.pad(edge_weight, (0, pad)).reshape(NW, NCH, C)
    col16 = col.reshape(NW, EPT // 16, 16)
    w16 = w.reshape(NW, EPT // 16, 16)

    degp = _deg_kernel(col16, w16)
    dinv = _tc0(degp)
    y1 = _tc1(x, W1, dinv)
    p = _agg_kernel(row, col, w, y1)
    y2 = _tc2(p, y1, dinv, b1, W2)
    q = _agg_kernel(row, col, w, y2)
    out = _tc3(q, y2, dinv, b2)
    return out


# speedup vs baseline: 1.8997x; 1.0456x over previous
import functools

import jax
import jax.numpy as jnp
import numpy as np
from jax import lax
from jax.experimental import pallas as pl
from jax.experimental.pallas import tpu as pltpu
from jax.experimental.pallas import tpu_sc as plsc

N = 10000
D = 128
E = 320000
NC = 2
NS = 16
NW = NC * NS
C = 64
NCH = 160
G = 80
NB = 4
EPT = NCH * C
EP = NW * EPT
RPT = N // NS

_mesh = plsc.VectorSubcoreMesh(core_axis_name="c", subcore_axis_name="s")
_sc_params = pltpu.CompilerParams(
    needs_layout_passes=False, use_tc_tiling_on_sc=False)


@functools.partial(
    pl.kernel,
    out_type=jax.ShapeDtypeStruct((NW, N), jnp.float32),
    mesh=_mesh,
    compiler_params=_sc_params,
    scratch_types=[
        pltpu.VMEM((EPT // 16, 16), jnp.int32),
        pltpu.VMEM((EPT // 16, 16), jnp.float32),
        pltpu.VMEM((N,), jnp.float32),
    ],
)
def _deg_kernel(col_h, w_h, out_h, col_v, w_v, deg_v):
    c = lax.axis_index("c")
    s = lax.axis_index("s")
    wid = c * NS + s
    zv = jnp.zeros((16,), jnp.float32)

    def zero(i, _):
        deg_v[pl.ds(i * 16, 16)] = zv
        return 0

    lax.fori_loop(0, N // 16, zero, 0)
    pltpu.sync_copy(col_h.at[wid], col_v)
    pltpu.sync_copy(w_h.at[wid], w_v)

    def body(i, _):
        plsc.addupdate_scatter(deg_v, [col_v[i]], w_v[i])
        return 0

    lax.fori_loop(0, EPT // 16, body, 0)
    pltpu.sync_copy(deg_v, out_h.at[wid])


@functools.partial(
    pl.kernel,
    out_type=jax.ShapeDtypeStruct((NC, N, D), jnp.float32),
    mesh=_mesh,
    compiler_params=_sc_params,
    scratch_types=[
        pltpu.VMEM((G, C), jnp.int32),
        pltpu.VMEM((G, C), jnp.int32),
        pltpu.VMEM((G, C), jnp.float32),
        [pltpu.VMEM((C, D), jnp.float32)] * NB,
        pltpu.VMEM_SHARED((N, D), jnp.float32),
        [pltpu.SemaphoreType.DMA] * NB,
        [pltpu.SemaphoreType.DMA] * NB,
    ],
)
def _agg_kernel(row_h, col_h, w_h, y_h, out_h,
                row_v, col_v, w_v, bufs, acc, gsems, ssems):
    c = lax.axis_index("c")
    s = lax.axis_index("s")
    wid = c * NS + s
    zv = jnp.zeros((16,), jnp.float32)

    def zrow(i, _):
        for j in range(D // 16):
            bufs[0][i, pl.ds(j * 16, 16)] = zv
        return 0

    lax.fori_loop(0, C, zrow, 0)

    def zacc(k, _):
        pltpu.sync_copy(bufs[0], acc.at[pl.ds(s * RPT + k * C, C)])
        return 0

    lax.fori_loop(0, RPT // C, zacc, 0)
    pltpu.sync_copy(bufs[0].at[pl.ds(0, RPT % C)],
                    acc.at[pl.ds(s * RPT + (RPT // C) * C, RPT % C)])
    plsc.subcore_barrier()

    def scale(buf, j):
        def sgrp(g16, _):
            wv = w_v[j, pl.ds(g16 * 16, 16)]
            for l in range(16):
                wb = jnp.take_along_axis(
                    wv, jnp.full((16,), l, jnp.int32), axis=0)
                for k in range(D // 16):
                    sl = pl.ds(k * 16, 16)
                    buf[g16 * 16 + l, sl] = buf[g16 * 16 + l, sl] * wb
            return 0

        lax.fori_loop(0, C // 16, sgrp, 0)

    def group(grp, _):
        pltpu.sync_copy(row_h.at[wid, pl.ds(grp * G, G)], row_v)
        pltpu.sync_copy(col_h.at[wid, pl.ds(grp * G, G)], col_v)
        pltpu.sync_copy(w_h.at[wid, pl.ds(grp * G, G)], w_v)
        for b in range(2):
            pltpu.async_copy(y_h.at[row_v.at[b]], bufs[b], gsems[b])

        def chunk4(q, _):
            for b in range(NB):
                j = q * NB + b
                buf = bufs[b]
                pltpu.make_async_copy(
                    y_h.at[row_v.at[j]], buf, gsems[b]).wait()
                scale(buf, j)
                pltpu.async_copy(
                    buf, acc.at[col_v.at[j]], ssems[b], add=True)
                b2 = (b + 2) % NB
                j2 = j + 2

                @pl.when(j >= 2)
                def _():
                    pltpu.make_async_copy(
                        bufs[b2], acc.at[col_v.at[0]], ssems[b2]).wait()

                @pl.when(j2 < G)
                def _():
                    pltpu.async_copy(
                        y_h.at[row_v.at[j2]], bufs[b2], gsems[b2])

            return 0

        lax.fori_loop(0, G // NB, chunk4, 0)
        pltpu.make_async_copy(
            bufs[2], acc.at[col_v.at[0]], ssems[2]).wait()
        pltpu.make_async_copy(
            bufs[3], acc.at[col_v.at[0]], ssems[3]).wait()
        return 0

    lax.fori_loop(0, NCH // G, group, 0)
    plsc.subcore_barrier()
    pltpu.sync_copy(acc.at[pl.ds(s * RPT, RPT)],
                    out_h.at[c, pl.ds(s * RPT, RPT)])


_BLK = 1000
_GRID = N // _BLK


def _tc0_body(degp_ref, dinv_ref):
    deg = jnp.sum(degp_ref[...], axis=0) + 1.0
    dinv_ref[...] = lax.rsqrt(deg)[:, None]


def _tc1_body(x_ref, w_ref, dinv_ref, y_ref):
    xw = jnp.dot(x_ref[...], w_ref[...], preferred_element_type=jnp.float32)
    y_ref[...] = xw * dinv_ref[...]


def _tc2_body(p_ref, y1_ref, dinv_ref, b1_ref, w2_ref, y2_ref):
    dinv = dinv_ref[...]
    agg = p_ref[0] + p_ref[1] + y1_ref[...]
    h = jnp.maximum(agg * dinv + b1_ref[...][None, :], 0.0)
    hw = jnp.dot(h, w2_ref[...], preferred_element_type=jnp.float32)
    y2_ref[...] = hw * dinv


def _tc3_body(q_ref, y2_ref, dinv_ref, b2_ref, out_ref):
    agg = q_ref[0] + q_ref[1] + y2_ref[...]
    out_ref[...] = jnp.maximum(
        agg * dinv_ref[...] + b2_ref[...][None, :], 0.0)


_tc0 = pl.pallas_call(
    _tc0_body,
    out_shape=jax.ShapeDtypeStruct((N, 1), jnp.float32),
)

_tc1 = pl.pallas_call(
    _tc1_body,
    grid=(_GRID,),
    in_specs=[
        pl.BlockSpec((_BLK, D), lambda i: (i, 0)),
        pl.BlockSpec((D, D), lambda i: (0, 0)),
        pl.BlockSpec((_BLK, 1), lambda i: (i, 0)),
    ],
    out_specs=pl.BlockSpec((_BLK, D), lambda i: (i, 0)),
    out_shape=jax.ShapeDtypeStruct((N, D), jnp.float32),
)

_tc2 = pl.pallas_call(
    _tc2_body,
    grid=(_GRID,),
    in_specs=[
        pl.BlockSpec((NC, _BLK, D), lambda i: (0, i, 0)),
        pl.BlockSpec((_BLK, D), lambda i: (i, 0)),
        pl.BlockSpec((_BLK, 1), lambda i: (i, 0)),
        pl.BlockSpec((D,), lambda i: (0,)),
        pl.BlockSpec((D, D), lambda i: (0, 0)),
    ],
    out_specs=pl.BlockSpec((_BLK, D), lambda i: (i, 0)),
    out_shape=jax.ShapeDtypeStruct((N, D), jnp.float32),
)

_tc3 = pl.pallas_call(
    _tc3_body,
    grid=(_GRID,),
    in_specs=[
        pl.BlockSpec((NC, _BLK, D), lambda i: (0, i, 0)),
        pl.BlockSpec((_BLK, D), lambda i: (i, 0)),
        pl.BlockSpec((_BLK, 1), lambda i: (i, 0)),
        pl.BlockSpec((D,), lambda i: (0,)),
    ],
    out_specs=pl.BlockSpec((_BLK, D), lambda i: (i, 0)),
    out_shape=jax.ShapeDtypeStruct((N, D), jnp.float32),
)


@jax.jit
def kernel(x, edge_index, edge_weight, W1, b1, W2, b2):
    pad = EP - E
    pad_idx = jnp.asarray((np.arange(pad) * 13) % N, dtype=jnp.int32)
    row = jnp.concatenate([edge_index[0], pad_idx]).reshape(NW, NCH, C)
    col = jnp.concatenate([edge_index[1], pad_idx]).reshape(NW, NCH, C)
    w = jnp.pad(edge_weight, (0, pad)).reshape(NW, NCH, C)
    col16 = col.reshape(NW, EPT // 16, 16)
    w16 = w.reshape(NW, EPT // 16, 16)

    degp = _deg_kernel(col16, w16)
    dinv = _tc0(degp)
    y1 = _tc1(x, W1, dinv)
    p = _agg_kernel(row, col, w, y1)
    y2 = _tc2(p, y1, dinv, b1, W2)
    q = _agg_kernel(row, col, w, y2)
    out = _tc3(q, y2, dinv, b2)
    return out


# speedup vs baseline: 1.9057x; 1.0031x over previous
import functools

import jax
import jax.numpy as jnp
import numpy as np
from jax import lax
from jax.experimental import pallas as pl
from jax.experimental.pallas import tpu as pltpu
from jax.experimental.pallas import tpu_sc as plsc

N = 10000
D = 128
E = 320000
NC = 2
NS = 16
NW = NC * NS
C = 64
NCH = 160
G = 80
NB = 4
EPT = NCH * C
EP = NW * EPT
RPT = N // NS

_mesh = plsc.VectorSubcoreMesh(core_axis_name="c", subcore_axis_name="s")
_sc_params = pltpu.CompilerParams(
    needs_layout_passes=False, use_tc_tiling_on_sc=False)


@functools.partial(
    pl.kernel,
    out_type=jax.ShapeDtypeStruct((NW, N), jnp.float32),
    mesh=_mesh,
    compiler_params=_sc_params,
    scratch_types=[
        pltpu.VMEM((E // NW // 16, 16), jnp.int32),
        pltpu.VMEM((E // NW // 16, 16), jnp.float32),
        pltpu.VMEM((N,), jnp.float32),
    ],
)
def _deg_kernel(col_h, w_h, out_h, col_v, w_v, deg_v):
    c = lax.axis_index("c")
    s = lax.axis_index("s")
    wid = c * NS + s
    zv = jnp.zeros((16,), jnp.float32)

    def zero(i, _):
        deg_v[pl.ds(i * 16, 16)] = zv
        return 0

    lax.fori_loop(0, N // 16, zero, 0)
    pltpu.sync_copy(col_h.at[wid], col_v)
    pltpu.sync_copy(w_h.at[wid], w_v)

    def body(i, _):
        plsc.addupdate_scatter(deg_v, [col_v[i]], w_v[i])
        return 0

    lax.fori_loop(0, E // NW // 16, body, 0)
    pltpu.sync_copy(deg_v, out_h.at[wid])


@functools.partial(
    pl.kernel,
    out_type=jax.ShapeDtypeStruct((NC, N, D), jnp.float32),
    mesh=_mesh,
    compiler_params=_sc_params,
    scratch_types=[
        pltpu.VMEM((G, C), jnp.int32),
        pltpu.VMEM((G, C), jnp.int32),
        pltpu.VMEM((G, C), jnp.float32),
        [pltpu.VMEM((C, D), jnp.float32)] * NB,
        pltpu.VMEM_SHARED((N, D), jnp.float32),
        [pltpu.SemaphoreType.DMA] * NB,
        [pltpu.SemaphoreType.DMA] * NB,
    ],
)
def _agg_kernel(row_h, col_h, w_h, y_h, out_h,
                row_v, col_v, w_v, bufs, acc, gsems, ssems):
    c = lax.axis_index("c")
    s = lax.axis_index("s")
    wid = c * NS + s
    zv = jnp.zeros((16,), jnp.float32)

    def zrow(i, _):
        for j in range(D // 16):
            bufs[0][i, pl.ds(j * 16, 16)] = zv
        return 0

    lax.fori_loop(0, C, zrow, 0)

    def zacc(k, _):
        pltpu.sync_copy(bufs[0], acc.at[pl.ds(s * RPT + k * C, C)])
        return 0

    lax.fori_loop(0, RPT // C, zacc, 0)
    pltpu.sync_copy(bufs[0].at[pl.ds(0, RPT % C)],
                    acc.at[pl.ds(s * RPT + (RPT // C) * C, RPT % C)])
    plsc.subcore_barrier()

    def scale(buf, j):
        def sgrp(g16, _):
            wv = w_v[j, pl.ds(g16 * 16, 16)]
            for l in range(16):
                wb = jnp.take_along_axis(
                    wv, jnp.full((16,), l, jnp.int32), axis=0)
                for k in range(D // 16):
                    sl = pl.ds(k * 16, 16)
                    buf[g16 * 16 + l, sl] = buf[g16 * 16 + l, sl] * wb
            return 0

        lax.fori_loop(0, C // 16, sgrp, 0)

    def group(grp, _):
        pltpu.sync_copy(row_h.at[wid, pl.ds(grp * G, G)], row_v)
        pltpu.sync_copy(col_h.at[wid, pl.ds(grp * G, G)], col_v)
        pltpu.sync_copy(w_h.at[wid, pl.ds(grp * G, G)], w_v)
        for b in range(2):
            pltpu.async_copy(y_h.at[row_v.at[b]], bufs[b], gsems[b])

        def chunk4(q, _):
            for b in range(NB):
                j = q * NB + b
                buf = bufs[b]
                pltpu.make_async_copy(
                    y_h.at[row_v.at[j]], buf, gsems[b]).wait()
                scale(buf, j)
                pltpu.async_copy(
                    buf, acc.at[col_v.at[j]], ssems[b], add=True)
                b2 = (b + 2) % NB
                j2 = j + 2

                @pl.when(j >= 2)
                def _():
                    pltpu.make_async_copy(
                        bufs[b2], acc.at[col_v.at[0]], ssems[b2]).wait()

                @pl.when(j2 < G)
                def _():
                    pltpu.async_copy(
                        y_h.at[row_v.at[j2]], bufs[b2], gsems[b2])

            return 0

        lax.fori_loop(0, G // NB, chunk4, 0)
        pltpu.make_async_copy(
            bufs[2], acc.at[col_v.at[0]], ssems[2]).wait()
        pltpu.make_async_copy(
            bufs[3], acc.at[col_v.at[0]], ssems[3]).wait()
        return 0

    lax.fori_loop(0, NCH // G, group, 0)
    plsc.subcore_barrier()
    pltpu.sync_copy(acc.at[pl.ds(s * RPT, RPT)],
                    out_h.at[c, pl.ds(s * RPT, RPT)])


_BLK = 1000
_GRID = N // _BLK


def _tc0_body(degp_ref, dinv_ref):
    deg = jnp.sum(degp_ref[...], axis=0) + 1.0
    dinv_ref[...] = lax.rsqrt(deg)[:, None]


def _tc1_body(x_ref, w_ref, dinv_ref, y_ref):
    xw = jnp.dot(x_ref[...], w_ref[...], preferred_element_type=jnp.float32)
    y_ref[...] = xw * dinv_ref[...]


def _tc2_body(p_ref, y1_ref, dinv_ref, b1_ref, w2_ref, y2_ref):
    dinv = dinv_ref[...]
    agg = p_ref[0] + p_ref[1] + y1_ref[...]
    h = jnp.maximum(agg * dinv + b1_ref[...][None, :], 0.0)
    hw = jnp.dot(h, w2_ref[...], preferred_element_type=jnp.float32)
    y2_ref[...] = hw * dinv


def _tc3_body(q_ref, y2_ref, dinv_ref, b2_ref, out_ref):
    agg = q_ref[0] + q_ref[1] + y2_ref[...]
    out_ref[...] = jnp.maximum(
        agg * dinv_ref[...] + b2_ref[...][None, :], 0.0)


_tc0 = pl.pallas_call(
    _tc0_body,
    out_shape=jax.ShapeDtypeStruct((N, 1), jnp.float32),
)

_tc1 = pl.pallas_call(
    _tc1_body,
    grid=(_GRID,),
    in_specs=[
        pl.BlockSpec((_BLK, D), lambda i: (i, 0)),
        pl.BlockSpec((D, D), lambda i: (0, 0)),
        pl.BlockSpec((_BLK, 1), lambda i: (i, 0)),
    ],
    out_specs=pl.BlockSpec((_BLK, D), lambda i: (i, 0)),
    out_shape=jax.ShapeDtypeStruct((N, D), jnp.float32),
)

_tc2 = pl.pallas_call(
    _tc2_body,
    grid=(_GRID,),
    in_specs=[
        pl.BlockSpec((NC, _BLK, D), lambda i: (0, i, 0)),
        pl.BlockSpec((_BLK, D), lambda i: (i, 0)),
        pl.BlockSpec((_BLK, 1), lambda i: (i, 0)),
        pl.BlockSpec((D,), lambda i: (0,)),
        pl.BlockSpec((D, D), lambda i: (0, 0)),
    ],
    out_specs=pl.BlockSpec((_BLK, D), lambda i: (i, 0)),
    out_shape=jax.ShapeDtypeStruct((N, D), jnp.float32),
)

_tc3 = pl.pallas_call(
    _tc3_body,
    grid=(_GRID,),
    in_specs=[
        pl.BlockSpec((NC, _BLK, D), lambda i: (0, i, 0)),
        pl.BlockSpec((_BLK, D), lambda i: (i, 0)),
        pl.BlockSpec((_BLK, 1), lambda i: (i, 0)),
        pl.BlockSpec((D,), lambda i: (0,)),
    ],
    out_specs=pl.BlockSpec((_BLK, D), lambda i: (i, 0)),
    out_shape=jax.ShapeDtypeStruct((N, D), jnp.float32),
)


@jax.jit
def kernel(x, edge_index, edge_weight, W1, b1, W2, b2):
    pad = EP - E
    pad_idx = jnp.asarray((np.arange(pad) * 13) % N, dtype=jnp.int32)
    row = jnp.concatenate([edge_index[0], pad_idx]).reshape(NW, NCH, C)
    col = jnp.concatenate([edge_index[1], pad_idx]).reshape(NW, NCH, C)
    w = jnp.pad(edge_weight, (0, pad)).reshape(NW, NCH, C)
    col16 = edge_index[1].reshape(NW, E // NW // 16, 16)
    w16 = edge_weight.reshape(NW, E // NW // 16, 16)

    degp = _deg_kernel(col16, w16)
    dinv = _tc0(degp)
    y1 = _tc1(x, W1, dinv)
    p = _agg_kernel(row, col, w, y1)
    y2 = _tc2(p, y1, dinv, b1, W2)
    q = _agg_kernel(row, col, w, y2)
    out = _tc3(q, y2, dinv, b2)
    return out
